# baseline (speedup 1.0000x reference)
import functools

import numpy as np
import jax
import jax.numpy as jnp
from jax import lax
from jax.experimental import pallas as pl
from jax.experimental.pallas import tpu as pltpu
from jax.experimental.pallas import tpu_sc as plsc

_DIAGONAL_OFFSET = 2
_L = 16


def _rc_order_np(ut_len: int, diagonal_offset: int) -> np.ndarray:
    seq_len = int(np.sqrt(2 * ut_len + 0.25) - 0.5) + diagonal_offset
    ut_indexes = np.triu_indices(seq_len, diagonal_offset)
    mat_ut_indexes = np.zeros((seq_len, seq_len), dtype=np.int64)
    mat_ut_indexes[ut_indexes] = np.arange(ut_len)
    mask_ut = np.zeros((seq_len, seq_len), dtype=bool)
    mask_ut[ut_indexes] = True
    mat_indexes = mat_ut_indexes + np.multiply(~mask_ut, mat_ut_indexes.T)
    return mat_indexes[::-1, ::-1][ut_indexes].astype(np.int32)


@functools.lru_cache(maxsize=4)
def _build(B: int, U: int, D: int):
    N = B * U
    NC, NS = 2, 16
    NW = NC * NS
    CH = (N // NW) // 8 * 8
    REM = N - NW * CH
    PIECE = 512
    n_full = CH // PIECE
    tail = CH - n_full * PIECE
    sizes = [PIECE] * n_full + ([tail] if tail else [])
    n_pieces = len(sizes)
    REM_BASE = NW * CH
    N_PAD = N if REM == 0 else REM_BASE + _L

    mesh = plsc.VectorSubcoreMesh(core_axis_name="c", subcore_axis_name="s")

    @functools.partial(
        pl.kernel,
        mesh=mesh,
        out_type=jax.ShapeDtypeStruct((N, D), jnp.float32),
        compiler_params=pltpu.CompilerParams(use_tc_tiling_on_sc=False),
        scratch_types=[
            pltpu.VMEM((CH,), jnp.int32),
            pltpu.VMEM((PIECE, D), jnp.float32),
            pltpu.VMEM((PIECE, D), jnp.float32),
            pltpu.VMEM((_L,), jnp.int32),
            pltpu.SemaphoreType.DMA,
            pltpu.SemaphoreType.DMA,
        ],
    )
    def sc_gather(x_hbm, rc_hbm, rev_hbm, out_hbm,
                  idx_v, buf0, buf1, rev_v, sem0, sem1):
        wid = lax.axis_index("s") * NC + lax.axis_index("c")
        base = wid * CH

        pltpu.sync_copy(rev_hbm, rev_v)
        m = rev_v[...] != 0
        lane = lax.iota(jnp.int32, _L)

        pltpu.sync_copy(rc_hbm.at[pl.ds(base, CH)], idx_v)

        def sel(i, c):
            off = pl.multiple_of(i * _L, _L)
            ident = base + off + lane
            idx_v[pl.ds(off, _L)] = jnp.where(m, idx_v[pl.ds(off, _L)], ident)
            return c

        lax.fori_loop(0, CH // _L, sel, 0)

        bufs = (buf0, buf1)
        sems = (sem0, sem1)

        def start(p):
            size = sizes[p]
            src = x_hbm.at[idx_v.at[pl.ds(p * PIECE, size)]]
            return pltpu.async_copy(src, bufs[p % 2].at[pl.ds(0, size)],
                                    sems[p % 2])

        copies = {0: start(0)}
        if n_pieces > 1:
            copies[1] = start(1)
        for p in range(n_pieces):
            copies[p].wait()
            size = sizes[p]
            pltpu.sync_copy(bufs[p % 2].at[pl.ds(0, size)],
                            out_hbm.at[pl.ds(base + p * PIECE, size)])
            if p + 2 < n_pieces:
                copies[p + 2] = start(p + 2)

        if REM:
            @pl.when(wid == NW - 1)
            def _():
                pltpu.sync_copy(rc_hbm.at[pl.ds(REM_BASE, _L)],
                                idx_v.at[pl.ds(0, _L)])
                ident = REM_BASE + lane
                idx_v[pl.ds(0, _L)] = jnp.where(m, idx_v[pl.ds(0, _L)], ident)
                cp = pltpu.async_copy(x_hbm.at[idx_v.at[pl.ds(0, REM)]],
                                      buf0.at[pl.ds(0, REM)], sem0)
                cp.wait()
                pltpu.sync_copy(buf0.at[pl.ds(0, REM)],
                                out_hbm.at[pl.ds(REM_BASE, REM)])

    rc = _rc_order_np(U, _DIAGONAL_OFFSET)
    rc_flat = (rc[None, :] + (np.arange(B, dtype=np.int64) * U)[:, None])
    rc_flat = rc_flat.reshape(-1).astype(np.int32)
    if N_PAD > N:
        rc_flat = np.concatenate([rc_flat, np.zeros(N_PAD - N, np.int32)])
    return sc_gather, jnp.asarray(rc_flat)


def kernel(x_ut, reverse):
    B, U, D = x_ut.shape
    sc_gather, rc_flat = _build(B, U, D)
    rev16 = jnp.broadcast_to(jnp.asarray(reverse, jnp.int32), (_L,))
    out = sc_gather(x_ut.reshape(B * U, D), rc_flat, rev16)
    return out.reshape(B, U, D)


# speedup vs baseline: 1.0011x; 1.0011x over previous
import functools

import numpy as np
import jax
import jax.numpy as jnp
from jax import lax
from jax.experimental import pallas as pl
from jax.experimental.pallas import tpu as pltpu
from jax.experimental.pallas import tpu_sc as plsc

_DIAGONAL_OFFSET = 2
_L = 16


def _rc_order_np(ut_len: int, diagonal_offset: int) -> np.ndarray:
    seq_len = int(np.sqrt(2 * ut_len + 0.25) - 0.5) + diagonal_offset
    ut_indexes = np.triu_indices(seq_len, diagonal_offset)
    mat_ut_indexes = np.zeros((seq_len, seq_len), dtype=np.int64)
    mat_ut_indexes[ut_indexes] = np.arange(ut_len)
    mask_ut = np.zeros((seq_len, seq_len), dtype=bool)
    mask_ut[ut_indexes] = True
    mat_indexes = mat_ut_indexes + np.multiply(~mask_ut, mat_ut_indexes.T)
    return mat_indexes[::-1, ::-1][ut_indexes].astype(np.int32)


@functools.lru_cache(maxsize=4)
def _build(B: int, U: int, D: int):
    N = B * U
    NC, NS = 2, 16
    NW = NC * NS
    CH = (N // NW) // 8 * 8
    REM = N - NW * CH
    PIECE = 512
    n_full = CH // PIECE
    tail = CH - n_full * PIECE
    sizes = [PIECE] * n_full + ([tail] if tail else [])
    n_pieces = len(sizes)
    REM_BASE = NW * CH
    N_PAD = N if REM == 0 else REM_BASE + _L

    mesh = plsc.VectorSubcoreMesh(core_axis_name="c", subcore_axis_name="s")

    @functools.partial(
        pl.kernel,
        mesh=mesh,
        out_type=jax.ShapeDtypeStruct((N, D), jnp.float32),
        compiler_params=pltpu.CompilerParams(
            use_tc_tiling_on_sc=False,
            skip_device_barrier=True,
            disable_bounds_checks=True,
            disable_semaphore_checks=True,
        ),
        scratch_types=[
            pltpu.VMEM((CH,), jnp.int32),
            pltpu.VMEM((PIECE, D), jnp.float32),
            pltpu.VMEM((PIECE, D), jnp.float32),
            pltpu.VMEM((_L,), jnp.int32),
            pltpu.SemaphoreType.DMA,
            pltpu.SemaphoreType.DMA,
        ],
    )
    def sc_gather(x_hbm, rc_hbm, rev_hbm, out_hbm,
                  idx_v, buf0, buf1, rev_v, sem0, sem1):
        wid = lax.axis_index("s") * NC + lax.axis_index("c")
        base = wid * CH

        pltpu.sync_copy(rev_hbm, rev_v)
        m = rev_v[...] != 0
        lane = lax.iota(jnp.int32, _L)

        pltpu.sync_copy(rc_hbm.at[pl.ds(base, CH)], idx_v)

        def sel(i, c):
            off = pl.multiple_of(i * _L, _L)
            ident = base + off + lane
            idx_v[pl.ds(off, _L)] = jnp.where(m, idx_v[pl.ds(off, _L)], ident)
            return c

        lax.fori_loop(0, CH // _L, sel, 0)

        bufs = (buf0, buf1)
        sems = (sem0, sem1)

        def start(p):
            size = sizes[p]
            src = x_hbm.at[idx_v.at[pl.ds(p * PIECE, size)]]
            return pltpu.async_copy(src, bufs[p % 2].at[pl.ds(0, size)],
                                    sems[p % 2])

        copies = {0: start(0)}
        if n_pieces > 1:
            copies[1] = start(1)
        for p in range(n_pieces):
            copies[p].wait()
            size = sizes[p]
            pltpu.sync_copy(bufs[p % 2].at[pl.ds(0, size)],
                            out_hbm.at[pl.ds(base + p * PIECE, size)])
            if p + 2 < n_pieces:
                copies[p + 2] = start(p + 2)

        if REM:
            @pl.when(wid == NW - 1)
            def _():
                pltpu.sync_copy(rc_hbm.at[pl.ds(REM_BASE, _L)],
                                idx_v.at[pl.ds(0, _L)])
                ident = REM_BASE + lane
                idx_v[pl.ds(0, _L)] = jnp.where(m, idx_v[pl.ds(0, _L)], ident)
                cp = pltpu.async_copy(x_hbm.at[idx_v.at[pl.ds(0, REM)]],
                                      buf0.at[pl.ds(0, REM)], sem0)
                cp.wait()
                pltpu.sync_copy(buf0.at[pl.ds(0, REM)],
                                out_hbm.at[pl.ds(REM_BASE, REM)])

    rc = _rc_order_np(U, _DIAGONAL_OFFSET)
    rc_flat = (rc[None, :] + (np.arange(B, dtype=np.int64) * U)[:, None])
    rc_flat = rc_flat.reshape(-1).astype(np.int32)
    if N_PAD > N:
        rc_flat = np.concatenate([rc_flat, np.zeros(N_PAD - N, np.int32)])
    return sc_gather, jnp.asarray(rc_flat)


def kernel(x_ut, reverse):
    B, U, D = x_ut.shape
    sc_gather, rc_flat = _build(B, U, D)
    rev16 = jnp.broadcast_to(jnp.asarray(reverse, jnp.int32), (_L,))
    out = sc_gather(x_ut.reshape(B * U, D), rc_flat, rev16)
    return out.reshape(B, U, D)


# speedup vs baseline: 3.7443x; 3.7403x over previous
import functools

import numpy as np
import jax
import jax.numpy as jnp
from jax import lax
from jax.experimental import pallas as pl
from jax.experimental.pallas import tpu as pltpu
from jax.experimental.pallas import tpu_sc as plsc

_DIAGONAL_OFFSET = 2
_L = 16


def _rc_order_np(ut_len: int, diagonal_offset: int) -> np.ndarray:
    seq_len = int(np.sqrt(2 * ut_len + 0.25) - 0.5) + diagonal_offset
    ut_indexes = np.triu_indices(seq_len, diagonal_offset)
    mat_ut_indexes = np.zeros((seq_len, seq_len), dtype=np.int64)
    mat_ut_indexes[ut_indexes] = np.arange(ut_len)
    mask_ut = np.zeros((seq_len, seq_len), dtype=bool)
    mask_ut[ut_indexes] = True
    mat_indexes = mat_ut_indexes + np.multiply(~mask_ut, mat_ut_indexes.T)
    return mat_indexes[::-1, ::-1][ut_indexes].astype(np.int32)


@functools.lru_cache(maxsize=4)
def _build(B: int, U: int, D: int):
    NC, NS = 2, 16
    BPC = B // NC
    U8 = -(-U // 8) * 8
    CH = (U // NS) // 8 * 8
    REM = U - NS * CH
    P = 128
    sizes = [P] * (CH // P) + ([CH % P] if CH % P else [])
    np_ = len(sizes)
    REM_BASE = NS * CH
    U_PAD = U if REM == 0 else REM_BASE + _L
    G = D // _L

    mesh = plsc.VectorSubcoreMesh(core_axis_name="c", subcore_axis_name="s")

    @functools.partial(
        pl.kernel,
        mesh=mesh,
        out_type=jax.ShapeDtypeStruct((B, U, D), jnp.float32),
        scratch_types=[
            pltpu.HBM((B * U8, 2 * D), jnp.float32),
            pltpu.VMEM((CH,), jnp.int32),
            pltpu.VMEM((P, D), jnp.float32),
            pltpu.VMEM((P, D), jnp.float32),
            pltpu.VMEM((P, 2 * D), jnp.float32),
            pltpu.VMEM((P, 2 * D), jnp.float32),
            pltpu.VMEM((P, 2 * D), jnp.float32),
            pltpu.VMEM((P, D), jnp.float32),
            pltpu.VMEM((_L,), jnp.int32),
            pltpu.SemaphoreType.DMA,
            pltpu.SemaphoreType.DMA,
        ],
    )
    def sc_gather(x_hbm, rc_hbm, rev_hbm, out_hbm,
                  lin_hbm, idx_v, la0, la1, lb, gb0, gb1, ob, rev_v,
                  sem0, sem1):
        cid = lax.axis_index("c")
        sid = lax.axis_index("s")
        base = sid * CH
        las, gbs, sems = (la0, la1), (gb0, gb1), (sem0, sem1)

        pltpu.sync_copy(rev_hbm, rev_v)
        m = rev_v[...] != 0
        lane = lax.iota(jnp.int32, _L)

        for q in range(BPC):
            b = cid * BPC + q
            sbase = b * U8 + base

            def startl(p):
                size = sizes[p]
                return pltpu.async_copy(
                    x_hbm.at[b, pl.ds(base + p * P, size), :],
                    las[p % 2].at[pl.ds(0, size)], sems[p % 2])

            def repack(src, size):
                def body(t, c):
                    for g in range(G):
                        lb[t, pl.ds(g * _L, _L)] = src[t, pl.ds(g * _L, _L)]
                    return c
                lax.fori_loop(0, size, body, 0)

            copies = {0: startl(0)}
            if np_ > 1:
                copies[1] = startl(1)
            for p in range(np_):
                copies[p].wait()
                size = sizes[p]
                if p + 2 < np_:
                    copies[p + 2] = startl(p + 2)
                repack(las[p % 2], size)
                pltpu.sync_copy(
                    lb.at[pl.ds(0, size)],
                    lin_hbm.at[pl.ds(sbase + p * P, size)])

            if REM:
                @pl.when(sid == NS - 1)
                def _():
                    cp = pltpu.async_copy(
                        x_hbm.at[b, pl.ds(REM_BASE, REM), :],
                        las[0].at[pl.ds(0, REM)], sems[0])
                    cp.wait()
                    repack(las[0], REM)
                    pltpu.sync_copy(
                        lb.at[pl.ds(0, REM)],
                        lin_hbm.at[pl.ds(b * U8 + REM_BASE, REM)])

        plsc.subcore_barrier()

        for q in range(BPC):
            b = cid * BPC + q
            obase = b * U8

            pltpu.sync_copy(rc_hbm.at[pl.ds(base, CH)], idx_v)

            def sel(i, c):
                off = pl.multiple_of(i * _L, _L)
                ident = base + off + lane
                idx_v[pl.ds(off, _L)] = obase + jnp.where(
                    m, idx_v[pl.ds(off, _L)], ident)
                return c

            lax.fori_loop(0, CH // _L, sel, 0)

            def startg(p):
                size = sizes[p]
                src = lin_hbm.at[idx_v.at[pl.ds(p * P, size)]]
                return pltpu.async_copy(
                    src, gbs[p % 2].at[pl.ds(0, size)], sems[p % 2])

            def compact(src, size):
                def body(t, c):
                    for g in range(G):
                        ob[t, pl.ds(g * _L, _L)] = src[t, pl.ds(g * _L, _L)]
                    return c
                lax.fori_loop(0, size, body, 0)

            copies = {0: startg(0)}
            if np_ > 1:
                copies[1] = startg(1)
            for p in range(np_):
                copies[p].wait()
                size = sizes[p]
                if p + 2 < np_:
                    copies[p + 2] = startg(p + 2)
                compact(gbs[p % 2], size)
                pltpu.sync_copy(
                    ob.at[pl.ds(0, size)],
                    out_hbm.at[b, pl.ds(base + p * P, size), :])

            if REM:
                @pl.when(sid == NS - 1)
                def _():
                    pltpu.sync_copy(rc_hbm.at[pl.ds(REM_BASE, _L)],
                                    idx_v.at[pl.ds(0, _L)])
                    ident = REM_BASE + lane
                    idx_v[pl.ds(0, _L)] = obase + jnp.where(
                        m, idx_v[pl.ds(0, _L)], ident)
                    cp = pltpu.async_copy(
                        lin_hbm.at[idx_v.at[pl.ds(0, REM)]],
                        gbs[0].at[pl.ds(0, REM)], sems[0])
                    cp.wait()
                    compact(gbs[0], REM)
                    pltpu.sync_copy(
                        ob.at[pl.ds(0, REM)],
                        out_hbm.at[b, pl.ds(REM_BASE, REM), :])

    rc = _rc_order_np(U, _DIAGONAL_OFFSET)
    if U_PAD > U:
        rc = np.concatenate([rc, np.zeros(U_PAD - U, np.int32)])
    return sc_gather, jnp.asarray(rc)


def kernel(x_ut, reverse):
    B, U, D = x_ut.shape
    sc_gather, rc = _build(B, U, D)
    rev16 = jnp.broadcast_to(jnp.asarray(reverse, jnp.int32), (_L,))
    return sc_gather(x_ut, rc, rev16)
